# Initial kernel scaffold; baseline (speedup 1.0000x reference)
import jax
import jax.numpy as jnp
from jax.experimental import pallas as pl


def kernel(src_nodes, edge_feats, edge_times, memory, last_update, time_w, time_b, W1, b1, W2, b2, W_ih, W_hh, b_ih, b_hh):
    raise NotImplementedError("write your pallas kernel here")


# speedup vs baseline: 1.1966x; 1.1966x over previous
import functools

import jax
import jax.numpy as jnp
from jax import lax
from jax.experimental import pallas as pl
from jax.experimental.pallas import tpu as pltpu
from jax.experimental.pallas import tpu_sc as plsc

B = 4096
MEM_DIM = 500
MP = 512
MSG_DIM = 100
EDGE_DIM = 16
TIME_DIM = 100
HID = (MEM_DIM + EDGE_DIM + 1 + TIME_DIM) // 2

NC, NS = 2, 16
NW = NC * NS
BPW = B // NW


def _sc_mesh():
    return plsc.VectorSubcoreMesh(core_axis_name="c", subcore_axis_name="s",
                                  num_cores=NC, num_subcores=NS)


_SC_PARAMS = pltpu.CompilerParams(use_tc_tiling_on_sc=False)


@functools.lru_cache(maxsize=None)
def _build_sc_gather_mem():
    @functools.partial(
        pl.kernel,
        out_type=(jax.ShapeDtypeStruct((B, MP), jnp.float32),
                  jax.ShapeDtypeStruct((B,), jnp.float32)),
        mesh=_sc_mesh(),
        scratch_types=[
            pltpu.VMEM((BPW,), jnp.int32),
            pltpu.VMEM((BPW, MP), jnp.float32),
            pltpu.VMEM((BPW,), jnp.float32),
            pltpu.SemaphoreType.DMA,
            pltpu.SemaphoreType.DMA,
        ],
        compiler_params=_SC_PARAMS,
    )
    def sc_gather_mem(mem_hbm, lu_hbm, idx_hbm, rows_out, lu_out,
                      idx_v, rows_v, lu_v, sem1, sem2):
        wid = lax.axis_index("s") * NC + lax.axis_index("c")
        base = wid * BPW
        pltpu.sync_copy(idx_hbm.at[pl.ds(base, BPW)], idx_v)
        c1 = pltpu.async_copy(mem_hbm.at[idx_v], rows_v, sem1)
        c2 = pltpu.async_copy(lu_hbm.at[idx_v], lu_v, sem2)
        c1.wait()
        c2.wait()
        pltpu.sync_copy(rows_v, rows_out.at[pl.ds(base, BPW)])
        pltpu.sync_copy(lu_v, lu_out.at[pl.ds(base, BPW)])

    return sc_gather_mem


@functools.lru_cache(maxsize=None)
def _build_sc_gather_hnew():
    @functools.partial(
        pl.kernel,
        out_type=jax.ShapeDtypeStruct((B, MP), jnp.float32),
        mesh=_sc_mesh(),
        scratch_types=[
            pltpu.VMEM((BPW,), jnp.int32),
            pltpu.VMEM((BPW, MP), jnp.float32),
            pltpu.SemaphoreType.DMA,
        ],
        compiler_params=_SC_PARAMS,
    )
    def sc_gather_hnew(hnew_hbm, idx_hbm, rows_out, idx_v, rows_v, sem1):
        wid = lax.axis_index("s") * NC + lax.axis_index("c")
        base = wid * BPW
        pltpu.sync_copy(idx_hbm.at[pl.ds(base, BPW)], idx_v)
        pltpu.async_copy(hnew_hbm.at[idx_v], rows_v, sem1).wait()
        pltpu.sync_copy(rows_v, rows_out.at[pl.ds(base, BPW)])

    return sc_gather_hnew


def _sc_gather_mem(mem, lu, idx):
    return _build_sc_gather_mem()(mem, lu, idx)


def _sc_gather_hnew(hnew, idx):
    return _build_sc_gather_hnew()(hnew, idx)


BB = 512
NBLK = B // BB
JCH = 1024


def _tc_body(hprev, ef, et3, lu3, src3, srcT, tw, tb,
             W1a, W1b, W1c, W1d, b1, W2, b2,
             Wihr, Wihz, Wihn, Whhr, Whhz, Whhn,
             br, bz, bin_, bhn,
             hnew_ref, w_ref):
    f32 = jnp.float32
    dg = functools.partial(lax.dot_general, preferred_element_type=f32)
    hp = hprev[...]
    dt = et3[0] - lu3[0]
    teT = jnp.cos(tw[...] * dt + tb[...])

    acc = dg(hp, W1a[...], (((1,), (0,)), ((), ())))
    acc = acc + dg(teT, W1d[...], (((0,), (0,)), ((), ())))
    acc = acc + dg(ef[...], W1b[...], (((1,), (0,)), ((), ())))
    acc = acc + dg(dt, W1c[...], (((0,), (0,)), ((), ())))
    h1 = jnp.maximum(acc + b1[...], 0.0)
    msg = dg(h1, W2[...], (((1,), (0,)), ((), ()))) + b2[...]

    gr = dg(msg, Wihr[...], (((1,), (0,)), ((), ()))) \
        + dg(hp, Whhr[...], (((1,), (0,)), ((), ()))) + br[...]
    gz = dg(msg, Wihz[...], (((1,), (0,)), ((), ()))) \
        + dg(hp, Whhz[...], (((1,), (0,)), ((), ()))) + bz[...]
    hn = dg(hp, Whhn[...], (((1,), (0,)), ((), ()))) + bhn[...]
    inn = dg(msg, Wihn[...], (((1,), (0,)), ((), ()))) + bin_[...]
    r = jax.nn.sigmoid(gr)
    z = jax.nn.sigmoid(gz)
    n = jnp.tanh(inn + r * hn)
    hnew_ref[...] = (1.0 - z) * n + z * hp

    si = src3[0]
    w = jnp.full((1, BB), -1, jnp.int32)
    for k in range(B // JCH):
        sj = srcT[pl.ds(k * JCH, JCH), :]
        jio = lax.broadcasted_iota(jnp.int32, (JCH, BB), 0) + (k * JCH)
        cand = jnp.where(sj == si, jio, -1)
        w = jnp.maximum(w, jnp.max(cand, axis=0, keepdims=True))
    w_ref[0] = w


def _tc_main(hprev, ef, et3, lu3, src3, srcT, tw, tb,
             W1a, W1b, W1c, W1d, b1, W2, b2,
             Wihr, Wihz, Wihn, Whhr, Whhz, Whhn,
             br, bz, bin_, bhn):
    row3 = pl.BlockSpec((1, 1, BB), lambda b: (b, 0, 0))

    def const2(shape):
        return pl.BlockSpec(shape, lambda b: (0, 0))

    in_specs = [
        pl.BlockSpec((BB, MP), lambda b: (b, 0)),
        pl.BlockSpec((BB, EDGE_DIM), lambda b: (b, 0)),
        row3,
        row3,
        row3,
        const2((B, 1)),
        const2((TIME_DIM, 1)),
        const2((TIME_DIM, 1)),
        const2((MP, HID)),
        const2((EDGE_DIM, HID)),
        const2((1, HID)),
        const2((TIME_DIM, HID)),
        const2((1, HID)),
        const2((HID, MSG_DIM)),
        const2((1, MSG_DIM)),
        const2((MSG_DIM, MP)),
        const2((MSG_DIM, MP)),
        const2((MSG_DIM, MP)),
        const2((MP, MP)),
        const2((MP, MP)),
        const2((MP, MP)),
        const2((1, MP)),
        const2((1, MP)),
        const2((1, MP)),
        const2((1, MP)),
    ]
    out_specs = [
        pl.BlockSpec((BB, MP), lambda b: (b, 0)),
        pl.BlockSpec((1, 1, BB), lambda b: (b, 0, 0)),
    ]
    out_shape = [
        jax.ShapeDtypeStruct((B, MP), jnp.float32),
        jax.ShapeDtypeStruct((NBLK, 1, BB), jnp.int32),
    ]
    return pl.pallas_call(
        _tc_body,
        grid=(NBLK,),
        in_specs=in_specs,
        out_specs=out_specs,
        out_shape=out_shape,
        compiler_params=pltpu.CompilerParams(
            dimension_semantics=("arbitrary",)),
    )(hprev, ef, et3, lu3, src3, srcT, tw, tb,
      W1a, W1b, W1c, W1d, b1, W2, b2,
      Wihr, Wihz, Wihn, Whhr, Whhz, Whhn, br, bz, bin_, bhn)


def kernel(src_nodes, edge_feats, edge_times, memory, last_update,
           time_w, time_b, W1, b1, W2, b2, W_ih, W_hh, b_ih, b_hh):
    f32 = jnp.float32
    src = src_nodes.astype(jnp.int32)
    M, PAD = MEM_DIM, MP - MEM_DIM

    mem512 = jnp.pad(memory, ((0, 0), (0, PAD)))
    hprev, lu = _sc_gather_mem(mem512, last_update, src)

    et3 = edge_times.reshape(NBLK, 1, BB)
    lu3 = lu.reshape(NBLK, 1, BB)
    src3 = src.reshape(NBLK, 1, BB)
    srcT = src.reshape(B, 1)
    tw = time_w.reshape(TIME_DIM, 1)
    tb = time_b.reshape(TIME_DIM, 1)

    W1a = jnp.pad(W1[:M], ((0, PAD), (0, 0)))
    W1b = W1[M:M + EDGE_DIM]
    W1c = W1[M + EDGE_DIM:M + EDGE_DIM + 1]
    W1d = W1[M + EDGE_DIM + 1:]
    b1r = b1.reshape(1, HID)
    b2r = b2.reshape(1, MSG_DIM)

    def padw(x):
        return jnp.pad(x, ((0, 0), (0, PAD)))

    def padhh(x):
        return jnp.pad(x, ((0, PAD), (0, PAD)))

    Wihr, Wihz, Wihn = (padw(W_ih[:, :M]), padw(W_ih[:, M:2 * M]),
                        padw(W_ih[:, 2 * M:]))
    Whhr, Whhz, Whhn = (padhh(W_hh[:, :M]), padhh(W_hh[:, M:2 * M]),
                        padhh(W_hh[:, 2 * M:]))

    def padb(x):
        return jnp.pad(x, (0, PAD)).reshape(1, MP)

    br = padb(b_ih[:M] + b_hh[:M]).astype(f32)
    bz = padb(b_ih[M:2 * M] + b_hh[M:2 * M]).astype(f32)
    bin_ = padb(b_ih[2 * M:])
    bhn = padb(b_hh[2 * M:])

    h_new, w3 = _tc_main(hprev, edge_feats, et3, lu3, src3, srcT, tw, tb,
                         W1a, W1b, W1c, W1d, b1r, W2, b2r,
                         Wihr, Wihz, Wihn, Whhr, Whhz, Whhn,
                         br, bz, bin_, bhn)

    out = _sc_gather_hnew(h_new, w3.reshape(B))
    return out[:, :M]


# speedup vs baseline: 5.4205x; 4.5299x over previous
import functools

import jax
import jax.numpy as jnp
from jax import lax
from jax.experimental import pallas as pl
from jax.experimental.pallas import tpu as pltpu
from jax.experimental.pallas import tpu_sc as plsc

B = 4096
MEM_DIM = 500
MSG_DIM = 100
EDGE_DIM = 16
TIME_DIM = 100
HID = (MEM_DIM + EDGE_DIM + 1 + TIME_DIM) // 2
G3 = 3 * MEM_DIM

NC, NS = 2, 16
NW = NC * NS
BPW = B // NW


def _sc_mesh():
    return plsc.VectorSubcoreMesh(core_axis_name="c", subcore_axis_name="s",
                                  num_cores=NC, num_subcores=NS)


_SC_PARAMS = pltpu.CompilerParams(use_tc_tiling_on_sc=True)


@functools.lru_cache(maxsize=None)
def _build_sc_gather_mem():
    @functools.partial(
        pl.kernel,
        out_type=(jax.ShapeDtypeStruct((B, MEM_DIM), jnp.float32),
                  jax.ShapeDtypeStruct((B,), jnp.float32)),
        mesh=_sc_mesh(),
        scratch_types=[
            pltpu.VMEM((BPW,), jnp.int32),
            pltpu.VMEM((BPW, MEM_DIM), jnp.float32),
            pltpu.VMEM((BPW,), jnp.float32),
            pltpu.SemaphoreType.DMA,
            pltpu.SemaphoreType.DMA,
        ],
        compiler_params=_SC_PARAMS,
    )
    def sc_gather_mem(mem_hbm, lu_hbm, idx_hbm, rows_out, lu_out,
                      idx_v, rows_v, lu_v, sem1, sem2):
        wid = lax.axis_index("s") * NC + lax.axis_index("c")
        base = wid * BPW
        pltpu.sync_copy(idx_hbm.at[pl.ds(base, BPW)], idx_v)
        c2 = pltpu.async_copy(lu_hbm.at[idx_v], lu_v, sem2)
        copies = []
        for g in range(BPW // 16):
            vec = idx_v[pl.ds(g * 16, 16)]
            for k in range(16):
                j = g * 16 + k
                copies.append(pltpu.async_copy(
                    mem_hbm.at[pl.ds(vec[k], 1)], rows_v.at[pl.ds(j, 1)],
                    sem1))
        for c in copies:
            c.wait()
        c2.wait()
        pltpu.sync_copy(rows_v, rows_out.at[pl.ds(base, BPW)])
        pltpu.sync_copy(lu_v, lu_out.at[pl.ds(base, BPW)])

    return sc_gather_mem


@functools.lru_cache(maxsize=None)
def _build_sc_gather_hnew():
    @functools.partial(
        pl.kernel,
        out_type=jax.ShapeDtypeStruct((B, MEM_DIM), jnp.float32),
        mesh=_sc_mesh(),
        scratch_types=[
            pltpu.VMEM((BPW,), jnp.int32),
            pltpu.VMEM((BPW, MEM_DIM), jnp.float32),
            pltpu.SemaphoreType.DMA,
        ],
        compiler_params=_SC_PARAMS,
    )
    def sc_gather_hnew(hnew_hbm, idx_hbm, rows_out, idx_v, rows_v, sem1):
        wid = lax.axis_index("s") * NC + lax.axis_index("c")
        base = wid * BPW
        pltpu.sync_copy(idx_hbm.at[pl.ds(base, BPW)], idx_v)
        copies = []
        for g in range(BPW // 16):
            vec = idx_v[pl.ds(g * 16, 16)]
            for k in range(16):
                j = g * 16 + k
                copies.append(pltpu.async_copy(
                    hnew_hbm.at[pl.ds(vec[k], 1)], rows_v.at[pl.ds(j, 1)],
                    sem1))
        for c in copies:
            c.wait()
        pltpu.sync_copy(rows_v, rows_out.at[pl.ds(base, BPW)])

    return sc_gather_hnew


def _sc_gather_mem(mem, lu, idx):
    return _build_sc_gather_mem()(mem, lu, idx)


def _sc_gather_hnew(hnew, idx):
    return _build_sc_gather_hnew()(hnew, idx)


BB = 512
NBLK = B // BB
JCH = 1024


def _tc_body(hprev, ef, et3, lu3, src3, srcT, tw, tb,
             W1a, W1b, W1c, W1d, b1, W2, b2,
             Wih, Whh, bih, bhh,
             hnew_ref, w_ref):
    f32 = jnp.float32
    M = MEM_DIM
    dg = functools.partial(lax.dot_general, preferred_element_type=f32)
    hp = hprev[...]
    dt = et3[0] - lu3[0]
    teT = jnp.cos(tw[...] * dt + tb[...])

    acc = dg(hp, W1a[...], (((1,), (0,)), ((), ())))
    acc = acc + dg(teT, W1d[...], (((0,), (0,)), ((), ())))
    acc = acc + dg(ef[...], W1b[...], (((1,), (0,)), ((), ())))
    acc = acc + dg(dt, W1c[...], (((0,), (0,)), ((), ())))
    h1 = jnp.maximum(acc + b1[...], 0.0)
    msg = dg(h1, W2[...], (((1,), (0,)), ((), ()))) + b2[...]

    gi = dg(msg, Wih[...], (((1,), (0,)), ((), ()))) + bih[...]
    gh = dg(hp, Whh[...], (((1,), (0,)), ((), ()))) + bhh[...]
    r = jax.nn.sigmoid(gi[:, :M] + gh[:, :M])
    z = jax.nn.sigmoid(gi[:, M:2 * M] + gh[:, M:2 * M])
    n = jnp.tanh(gi[:, 2 * M:] + r * gh[:, 2 * M:])
    hnew_ref[...] = (1.0 - z) * n + z * hp

    si = src3[0]
    w = jnp.full((1, BB), -1, jnp.int32)
    for k in range(B // JCH):
        sj = srcT[pl.ds(k * JCH, JCH), :]
        jio = lax.broadcasted_iota(jnp.int32, (JCH, BB), 0) + (k * JCH)
        cand = jnp.where(sj == si, jio, -1)
        w = jnp.maximum(w, jnp.max(cand, axis=0, keepdims=True))
    w_ref[0] = w


def _tc_main(hprev, ef, et3, lu3, src3, srcT, tw, tb,
             W1a, W1b, W1c, W1d, b1, W2, b2, Wih, Whh, bih, bhh):
    row3 = pl.BlockSpec((1, 1, BB), lambda b: (b, 0, 0))

    def const2(shape):
        return pl.BlockSpec(shape, lambda b: (0, 0))

    in_specs = [
        pl.BlockSpec((BB, MEM_DIM), lambda b: (b, 0)),
        pl.BlockSpec((BB, EDGE_DIM), lambda b: (b, 0)),
        row3,
        row3,
        row3,
        const2((B, 1)),
        const2((TIME_DIM, 1)),
        const2((TIME_DIM, 1)),
        const2((MEM_DIM, HID)),
        const2((EDGE_DIM, HID)),
        const2((1, HID)),
        const2((TIME_DIM, HID)),
        const2((1, HID)),
        const2((HID, MSG_DIM)),
        const2((1, MSG_DIM)),
        const2((MSG_DIM, G3)),
        const2((MEM_DIM, G3)),
        const2((1, G3)),
        const2((1, G3)),
    ]
    out_specs = [
        pl.BlockSpec((BB, MEM_DIM), lambda b: (b, 0)),
        pl.BlockSpec((1, 1, BB), lambda b: (b, 0, 0)),
    ]
    out_shape = [
        jax.ShapeDtypeStruct((B, MEM_DIM), jnp.float32),
        jax.ShapeDtypeStruct((NBLK, 1, BB), jnp.int32),
    ]
    return pl.pallas_call(
        _tc_body,
        grid=(NBLK,),
        in_specs=in_specs,
        out_specs=out_specs,
        out_shape=out_shape,
        compiler_params=pltpu.CompilerParams(
            dimension_semantics=("arbitrary",)),
    )(hprev, ef, et3, lu3, src3, srcT, tw, tb,
      W1a, W1b, W1c, W1d, b1, W2, b2, Wih, Whh, bih, bhh)


def kernel(src_nodes, edge_feats, edge_times, memory, last_update,
           time_w, time_b, W1, b1, W2, b2, W_ih, W_hh, b_ih, b_hh):
    src = src_nodes.astype(jnp.int32)
    M = MEM_DIM

    hprev, lu = _sc_gather_mem(memory, last_update, src)

    et3 = edge_times.reshape(NBLK, 1, BB)
    lu3 = lu.reshape(NBLK, 1, BB)
    src3 = src.reshape(NBLK, 1, BB)
    srcT = src.reshape(B, 1)
    tw = time_w.reshape(TIME_DIM, 1)
    tb = time_b.reshape(TIME_DIM, 1)

    W1a = W1[:M]
    W1b = W1[M:M + EDGE_DIM]
    W1c = W1[M + EDGE_DIM:M + EDGE_DIM + 1]
    W1d = W1[M + EDGE_DIM + 1:]
    b1r = b1.reshape(1, HID)
    b2r = b2.reshape(1, MSG_DIM)
    bih = b_ih.reshape(1, G3)
    bhh = b_hh.reshape(1, G3)

    h_new, w3 = _tc_main(hprev, edge_feats, et3, lu3, src3, srcT, tw, tb,
                         W1a, W1b, W1c, W1d, b1r, W2, b2r,
                         W_ih, W_hh, bih, bhh)

    return _sc_gather_hnew(h_new, w3.reshape(B))


# speedup vs baseline: 5.4218x; 1.0002x over previous
import functools

import jax
import jax.numpy as jnp
from jax import lax
from jax.experimental import pallas as pl
from jax.experimental.pallas import tpu as pltpu
from jax.experimental.pallas import tpu_sc as plsc

B = 4096
MEM_DIM = 500
MSG_DIM = 100
EDGE_DIM = 16
TIME_DIM = 100
RAW_DIM = MEM_DIM + EDGE_DIM + 1 + TIME_DIM
HID = RAW_DIM // 2
G3 = 3 * MEM_DIM

NC, NS = 2, 16
NW = NC * NS
BPW = B // NW


def _sc_mesh():
    return plsc.VectorSubcoreMesh(core_axis_name="c", subcore_axis_name="s",
                                  num_cores=NC, num_subcores=NS)


_SC_PARAMS = pltpu.CompilerParams(use_tc_tiling_on_sc=True)


@functools.lru_cache(maxsize=None)
def _build_sc_gather_mem():
    @functools.partial(
        pl.kernel,
        out_type=(jax.ShapeDtypeStruct((B, MEM_DIM), jnp.float32),
                  jax.ShapeDtypeStruct((B,), jnp.float32)),
        mesh=_sc_mesh(),
        scratch_types=[
            pltpu.VMEM((BPW,), jnp.int32),
            pltpu.VMEM((BPW, MEM_DIM), jnp.float32),
            pltpu.VMEM((BPW,), jnp.float32),
            pltpu.SemaphoreType.DMA,
            pltpu.SemaphoreType.DMA,
        ],
        compiler_params=_SC_PARAMS,
    )
    def sc_gather_mem(mem_hbm, lu_hbm, idx_hbm, rows_out, lu_out,
                      idx_v, rows_v, lu_v, sem1, sem2):
        wid = lax.axis_index("s") * NC + lax.axis_index("c")
        base = wid * BPW
        pltpu.sync_copy(idx_hbm.at[pl.ds(base, BPW)], idx_v)
        c2 = pltpu.async_copy(lu_hbm.at[idx_v], lu_v, sem2)
        copies = []
        for g in range(BPW // 16):
            vec = idx_v[pl.ds(g * 16, 16)]
            for k in range(16):
                j = g * 16 + k
                copies.append(pltpu.async_copy(
                    mem_hbm.at[pl.ds(vec[k], 1)], rows_v.at[pl.ds(j, 1)],
                    sem1))
        for c in copies:
            c.wait()
        c2.wait()
        pltpu.sync_copy(rows_v, rows_out.at[pl.ds(base, BPW)])
        pltpu.sync_copy(lu_v, lu_out.at[pl.ds(base, BPW)])

    return sc_gather_mem


@functools.lru_cache(maxsize=None)
def _build_sc_gather_hnew():
    @functools.partial(
        pl.kernel,
        out_type=jax.ShapeDtypeStruct((B, MEM_DIM), jnp.float32),
        mesh=_sc_mesh(),
        scratch_types=[
            pltpu.VMEM((BPW,), jnp.int32),
            pltpu.VMEM((BPW, MEM_DIM), jnp.float32),
            pltpu.SemaphoreType.DMA,
        ],
        compiler_params=_SC_PARAMS,
    )
    def sc_gather_hnew(hnew_hbm, idx_hbm, rows_out, idx_v, rows_v, sem1):
        wid = lax.axis_index("s") * NC + lax.axis_index("c")
        base = wid * BPW
        pltpu.sync_copy(idx_hbm.at[pl.ds(base, BPW)], idx_v)
        copies = []
        for g in range(BPW // 16):
            vec = idx_v[pl.ds(g * 16, 16)]
            for k in range(16):
                j = g * 16 + k
                copies.append(pltpu.async_copy(
                    hnew_hbm.at[pl.ds(vec[k], 1)], rows_v.at[pl.ds(j, 1)],
                    sem1))
        for c in copies:
            c.wait()
        pltpu.sync_copy(rows_v, rows_out.at[pl.ds(base, BPW)])

    return sc_gather_hnew


def _sc_gather_mem(mem, lu, idx):
    return _build_sc_gather_mem()(mem, lu, idx)


def _sc_gather_hnew(hnew, idx):
    return _build_sc_gather_hnew()(hnew, idx)


BB = 512
NBLK = B // BB
JCH = 1024


def _tc_body(hprev, ef, et, lu, src, tw, tb,
             W1, b1, W2, b2, Wih, Whh, bih, bhh,
             hnew_ref, w_ref):
    f32 = jnp.float32
    M = MEM_DIM
    b = pl.program_id(0)
    dg = functools.partial(lax.dot_general, preferred_element_type=f32)
    hp = hprev[...]
    dt = (et[pl.ds(b * BB, BB)] - lu[pl.ds(b * BB, BB)]).reshape(1, BB)
    twc = tw[...].reshape(TIME_DIM, 1)
    tbc = tb[...].reshape(TIME_DIM, 1)
    teT = jnp.cos(twc * dt + tbc)

    W1all = W1[...]
    acc = dg(hp, W1all[:M], (((1,), (0,)), ((), ())))
    acc = acc + dg(teT, W1all[M + EDGE_DIM + 1:], (((0,), (0,)), ((), ())))
    acc = acc + dg(ef[...], W1all[M:M + EDGE_DIM], (((1,), (0,)), ((), ())))
    acc = acc + dg(dt, W1all[M + EDGE_DIM:M + EDGE_DIM + 1],
                   (((0,), (0,)), ((), ())))
    h1 = jnp.maximum(acc + b1[...].reshape(1, HID), 0.0)
    msg = dg(h1, W2[...], (((1,), (0,)), ((), ()))) \
        + b2[...].reshape(1, MSG_DIM)

    gi = dg(msg, Wih[...], (((1,), (0,)), ((), ()))) \
        + bih[...].reshape(1, G3)
    gh = dg(hp, Whh[...], (((1,), (0,)), ((), ()))) \
        + bhh[...].reshape(1, G3)
    r = jax.nn.sigmoid(gi[:, :M] + gh[:, :M])
    z = jax.nn.sigmoid(gi[:, M:2 * M] + gh[:, M:2 * M])
    n = jnp.tanh(gi[:, 2 * M:] + r * gh[:, 2 * M:])
    hnew_ref[...] = (1.0 - z) * n + z * hp

    si = src[pl.ds(b * BB, BB)].reshape(1, BB)
    sjc = src[...].reshape(1, B)
    sjc = jnp.transpose(sjc, (1, 0))
    w = jnp.full((1, BB), -1, jnp.int32)
    for k in range(B // JCH):
        sj = sjc[k * JCH:(k + 1) * JCH, :]
        jio = lax.broadcasted_iota(jnp.int32, (JCH, BB), 0) + (k * JCH)
        cand = jnp.where(sj == si, jio, -1)
        w = jnp.maximum(w, jnp.max(cand, axis=0, keepdims=True))
    w_ref[...] = w.reshape(BB)


def _tc_main(hprev, ef, et, lu, src, tw, tb,
             W1, b1, W2, b2, Wih, Whh, bih, bhh):
    def full1(n):
        return pl.BlockSpec((n,), lambda b: (0,))

    def const2(shape):
        return pl.BlockSpec(shape, lambda b: (0, 0))

    in_specs = [
        pl.BlockSpec((BB, MEM_DIM), lambda b: (b, 0)),
        pl.BlockSpec((BB, EDGE_DIM), lambda b: (b, 0)),
        full1(B),
        full1(B),
        full1(B),
        full1(TIME_DIM),
        full1(TIME_DIM),
        const2((RAW_DIM, HID)),
        full1(HID),
        const2((HID, MSG_DIM)),
        full1(MSG_DIM),
        const2((MSG_DIM, G3)),
        const2((MEM_DIM, G3)),
        full1(G3),
        full1(G3),
    ]
    out_specs = [
        pl.BlockSpec((BB, MEM_DIM), lambda b: (b, 0)),
        pl.BlockSpec((BB,), lambda b: (b,)),
    ]
    out_shape = [
        jax.ShapeDtypeStruct((B, MEM_DIM), jnp.float32),
        jax.ShapeDtypeStruct((B,), jnp.int32),
    ]
    return pl.pallas_call(
        _tc_body,
        grid=(NBLK,),
        in_specs=in_specs,
        out_specs=out_specs,
        out_shape=out_shape,
        compiler_params=pltpu.CompilerParams(
            dimension_semantics=("arbitrary",)),
    )(hprev, ef, et, lu, src, tw, tb,
      W1, b1, W2, b2, Wih, Whh, bih, bhh)


def kernel(src_nodes, edge_feats, edge_times, memory, last_update,
           time_w, time_b, W1, b1, W2, b2, W_ih, W_hh, b_ih, b_hh):
    src = src_nodes.astype(jnp.int32)
    hprev, lu = _sc_gather_mem(memory, last_update, src)
    h_new, w = _tc_main(hprev, edge_feats, edge_times, lu, src, time_w,
                        time_b, W1, b1, W2, b2, W_ih, W_hh, b_ih, b_hh)
    return _sc_gather_hnew(h_new, w)


# speedup vs baseline: 5.4475x; 1.0047x over previous
import functools

import jax
import jax.numpy as jnp
from jax import lax
from jax.experimental import pallas as pl
from jax.experimental.pallas import tpu as pltpu
from jax.experimental.pallas import tpu_sc as plsc

B = 4096
MEM_DIM = 500
MSG_DIM = 100
EDGE_DIM = 16
TIME_DIM = 100
RAW_DIM = MEM_DIM + EDGE_DIM + 1 + TIME_DIM
HID = RAW_DIM // 2
G3 = 3 * MEM_DIM

NC, NS = 2, 16
NW = NC * NS
BPW = B // NW


def _sc_mesh():
    return plsc.VectorSubcoreMesh(core_axis_name="c", subcore_axis_name="s",
                                  num_cores=NC, num_subcores=NS)


_SC_PARAMS = pltpu.CompilerParams(use_tc_tiling_on_sc=True)


@functools.lru_cache(maxsize=None)
def _build_sc_gather_mem():
    @functools.partial(
        pl.kernel,
        out_type=(jax.ShapeDtypeStruct((B, 384), jnp.float32),
                  jax.ShapeDtypeStruct((B, 116), jnp.float32),
                  jax.ShapeDtypeStruct((B,), jnp.float32)),
        mesh=_sc_mesh(),
        scratch_types=[
            pltpu.VMEM((BPW,), jnp.int32),
            pltpu.VMEM((BPW, 384), jnp.float32),
            pltpu.VMEM((BPW, 116), jnp.float32),
            pltpu.VMEM((BPW,), jnp.float32),
            pltpu.SemaphoreType.DMA,
            pltpu.SemaphoreType.DMA,
            pltpu.SemaphoreType.DMA,
        ],
        compiler_params=_SC_PARAMS,
    )
    def sc_gather_mem(mem_hbm, lu_hbm, idx_hbm, rowsa_out, rowsb_out,
                      lu_out, idx_v, rowsa_v, rowsb_v, lu_v,
                      sem1, sem2, sem3):
        wid = lax.axis_index("s") * NC + lax.axis_index("c")
        base = wid * BPW
        pltpu.sync_copy(idx_hbm.at[pl.ds(base, BPW)], idx_v)
        c1 = pltpu.async_copy(mem_hbm.at[idx_v, pl.ds(0, 384)], rowsa_v,
                              sem1)
        c3 = pltpu.async_copy(lu_hbm.at[idx_v], lu_v, sem3)
        copies = []
        for g in range(BPW // 16):
            vec = idx_v[pl.ds(g * 16, 16)]
            for k in range(16):
                j = g * 16 + k
                copies.append(pltpu.async_copy(
                    mem_hbm.at[pl.ds(vec[k], 1), pl.ds(384, 116)],
                    rowsb_v.at[pl.ds(j, 1)], sem2))
        c1.wait()
        for c in copies:
            c.wait()
        c3.wait()
        pltpu.sync_copy(rowsa_v, rowsa_out.at[pl.ds(base, BPW)])
        pltpu.sync_copy(rowsb_v, rowsb_out.at[pl.ds(base, BPW)])
        pltpu.sync_copy(lu_v, lu_out.at[pl.ds(base, BPW)])

    return sc_gather_mem


@functools.lru_cache(maxsize=None)
def _build_sc_gather_hnew():
    @functools.partial(
        pl.kernel,
        out_type=jax.ShapeDtypeStruct((B, MEM_DIM), jnp.float32),
        mesh=_sc_mesh(),
        scratch_types=[
            pltpu.VMEM((BPW,), jnp.int32),
            pltpu.VMEM((BPW, MEM_DIM), jnp.float32),
            pltpu.SemaphoreType.DMA,
        ],
        compiler_params=_SC_PARAMS,
    )
    def sc_gather_hnew(hnew_hbm, idx_hbm, rows_out, idx_v, rows_v, sem1):
        wid = lax.axis_index("s") * NC + lax.axis_index("c")
        base = wid * BPW
        pltpu.sync_copy(idx_hbm.at[pl.ds(base, BPW)], idx_v)
        copies = []
        for g in range(BPW // 16):
            vec = idx_v[pl.ds(g * 16, 16)]
            for k in range(16):
                j = g * 16 + k
                copies.append(pltpu.async_copy(
                    hnew_hbm.at[pl.ds(vec[k], 1)], rows_v.at[pl.ds(j, 1)],
                    sem1))
        for c in copies:
            c.wait()
        pltpu.sync_copy(rows_v, rows_out.at[pl.ds(base, BPW)])

    return sc_gather_hnew


def _sc_gather_mem(mem, lu, idx):
    return _build_sc_gather_mem()(mem, lu, idx)


def _sc_gather_hnew(hnew, idx):
    return _build_sc_gather_hnew()(hnew, idx)


BB = 512
NBLK = B // BB
JCH = 1024


def _tc_body(hpa, hpb, ef, et, lu, src, tw, tb,
             W1, b1, W2, b2, Wih, Whh, bih, bhh,
             hnew_ref, w_ref):
    f32 = jnp.float32
    M = MEM_DIM
    b = pl.program_id(0)
    dg = functools.partial(lax.dot_general, preferred_element_type=f32)
    hp = jnp.concatenate([hpa[...], hpb[...]], axis=1)
    dt = (et[pl.ds(b * BB, BB)] - lu[pl.ds(b * BB, BB)]).reshape(1, BB)
    twc = tw[...].reshape(TIME_DIM, 1)
    tbc = tb[...].reshape(TIME_DIM, 1)
    teT = jnp.cos(twc * dt + tbc)

    W1all = W1[...]
    acc = dg(hp, W1all[:M], (((1,), (0,)), ((), ())))
    acc = acc + dg(teT, W1all[M + EDGE_DIM + 1:], (((0,), (0,)), ((), ())))
    acc = acc + dg(ef[...], W1all[M:M + EDGE_DIM], (((1,), (0,)), ((), ())))
    acc = acc + dg(dt, W1all[M + EDGE_DIM:M + EDGE_DIM + 1],
                   (((0,), (0,)), ((), ())))
    h1 = jnp.maximum(acc + b1[...].reshape(1, HID), 0.0)
    msg = dg(h1, W2[...], (((1,), (0,)), ((), ()))) \
        + b2[...].reshape(1, MSG_DIM)

    gi = dg(msg, Wih[...], (((1,), (0,)), ((), ()))) \
        + bih[...].reshape(1, G3)
    gh = dg(hp, Whh[...], (((1,), (0,)), ((), ()))) \
        + bhh[...].reshape(1, G3)
    r = jax.nn.sigmoid(gi[:, :M] + gh[:, :M])
    z = jax.nn.sigmoid(gi[:, M:2 * M] + gh[:, M:2 * M])
    n = jnp.tanh(gi[:, 2 * M:] + r * gh[:, 2 * M:])
    hnew_ref[...] = (1.0 - z) * n + z * hp

    si = src[pl.ds(b * BB, BB)].reshape(1, BB)
    sjc = src[...].reshape(1, B)
    sjc = jnp.transpose(sjc, (1, 0))
    w = jnp.full((1, BB), -1, jnp.int32)
    for k in range(B // JCH):
        sj = sjc[k * JCH:(k + 1) * JCH, :]
        jio = lax.broadcasted_iota(jnp.int32, (JCH, BB), 0) + (k * JCH)
        cand = jnp.where(sj == si, jio, -1)
        w = jnp.maximum(w, jnp.max(cand, axis=0, keepdims=True))
    w_ref[...] = w.reshape(BB)


def _tc_main(hpa, hpb, ef, et, lu, src, tw, tb,
             W1, b1, W2, b2, Wih, Whh, bih, bhh):
    def full1(n):
        return pl.BlockSpec((n,), lambda b: (0,))

    def const2(shape):
        return pl.BlockSpec(shape, lambda b: (0, 0))

    in_specs = [
        pl.BlockSpec((BB, 384), lambda b: (b, 0)),
        pl.BlockSpec((BB, 116), lambda b: (b, 0)),
        pl.BlockSpec((BB, EDGE_DIM), lambda b: (b, 0)),
        full1(B),
        full1(B),
        full1(B),
        full1(TIME_DIM),
        full1(TIME_DIM),
        const2((RAW_DIM, HID)),
        full1(HID),
        const2((HID, MSG_DIM)),
        full1(MSG_DIM),
        const2((MSG_DIM, G3)),
        const2((MEM_DIM, G3)),
        full1(G3),
        full1(G3),
    ]
    out_specs = [
        pl.BlockSpec((BB, MEM_DIM), lambda b: (b, 0)),
        pl.BlockSpec((BB,), lambda b: (b,)),
    ]
    out_shape = [
        jax.ShapeDtypeStruct((B, MEM_DIM), jnp.float32),
        jax.ShapeDtypeStruct((B,), jnp.int32),
    ]
    return pl.pallas_call(
        _tc_body,
        grid=(NBLK,),
        in_specs=in_specs,
        out_specs=out_specs,
        out_shape=out_shape,
        compiler_params=pltpu.CompilerParams(
            dimension_semantics=("arbitrary",)),
    )(hpa, hpb, ef, et, lu, src, tw, tb,
      W1, b1, W2, b2, Wih, Whh, bih, bhh)


def kernel(src_nodes, edge_feats, edge_times, memory, last_update,
           time_w, time_b, W1, b1, W2, b2, W_ih, W_hh, b_ih, b_hh):
    src = src_nodes.astype(jnp.int32)
    hpa, hpb, lu = _sc_gather_mem(memory, last_update, src)
    h_new, w = _tc_main(hpa, hpb, edge_feats, edge_times, lu, src, time_w,
                        time_b, W1, b1, W2, b2, W_ih, W_hh, b_ih, b_hh)
    return _sc_gather_hnew(h_new, w)
